# Initial kernel scaffold; baseline (speedup 1.0000x reference)
import jax
import jax.numpy as jnp
from jax.experimental import pallas as pl


def kernel(x, pe):
    raise NotImplementedError("write your pallas kernel here")


# speedup vs baseline: 1.4572x; 1.4572x over previous
import jax
import jax.numpy as jnp
from jax.experimental import pallas as pl


_TL = 256


def _add_kernel(x_ref, pe_ref, o_ref):
    o_ref[...] = x_ref[...] + pe_ref[...][None]


def kernel(x, pe):
    B, L, D = x.shape
    grid = (L // _TL, B)
    return pl.pallas_call(
        _add_kernel,
        grid=grid,
        in_specs=[
            pl.BlockSpec((1, _TL, D), lambda l, b: (b, l, 0)),
            pl.BlockSpec((_TL, D), lambda l, b: (l, 0)),
        ],
        out_specs=pl.BlockSpec((1, _TL, D), lambda l, b: (b, l, 0)),
        out_shape=jax.ShapeDtypeStruct((B, L, D), x.dtype),
    )(x, pe)
